# baseline (speedup 1.0000x reference)
import jax
import jax.numpy as jnp
from jax.experimental import pallas as pl
from jax.experimental.pallas import tpu as pltpu

_FIG = 640.0
_CONF_THRESH = 0.2
_B, _N = 8, 20000
_NB = 2560
_STEPS = (_N + _NB - 1) // _NB
_TAIL = _N - (_STEPS - 1) * _NB
_NEG_INF = float("-inf")


def _body(xa_ref, xb_ref, thr_ref, p_ref, det_ref, mp_ref, acc_ref):
    i = pl.program_id(0)

    gx1 = p_ref[:, 0:1]
    gy1 = p_ref[:, 1:2]
    gx2 = p_ref[:, 2:3]
    gy2 = p_ref[:, 3:4]
    tv = p_ref[:, 4:5]
    area2 = (gx2 - gx1) * (gy2 - gy1)

    cx = xa_ref[0]
    cy = xa_ref[1]
    w = xa_ref[2]
    h = xa_ref[3]
    conf = xa_ref[4]
    clsf = xb_ref[0]
    thr = thr_ref[0]

    hw = w * 0.5
    hh = h * 0.5
    w1 = cx - hw
    w2 = cx + hw
    h1 = cy - hh
    h2 = cy + hh
    iw = jnp.maximum(jnp.minimum(w2, gx2) - jnp.maximum(w1, gx1), 0.0)
    ih = jnp.maximum(jnp.minimum(h2, gy2) - jnp.maximum(h1, gy1), 0.0)
    inter = iw * ih
    area1 = (w2 - w1) * (h2 - h1)
    union = area1 + area2 - inter
    m = (inter >= tv * union) & (union > 0.0)
    m = m & (clsf.astype(jnp.int32) == 0) & (conf > thr)
    cand = jnp.where(m, conf, _NEG_INF)

    @pl.when(i == 0)
    def _():
        acc_ref[...] = cand

    @pl.when(i > 0)
    def _():
        acc_ref[...] = jnp.maximum(acc_ref[...], cand)

    @pl.when(i == _STEPS - 1)
    def _():
        mx = jnp.max(acc_ref[...], axis=1, keepdims=True)
        mp = jnp.where(mx == _NEG_INF, 0.0, mx)
        mp_ref[...] = mp
        det_ref[...] = jnp.broadcast_to(jnp.sum(mp) * (1.0 / _B), (1, 1))


def kernel(YOLOoutputs, gt, iou_thresh):
    xt = jnp.transpose(YOLOoutputs, (2, 0, 1))
    xa = xt[:5]
    xb = xt[6:7]
    gtn = gt * jnp.float32(1.0 / _FIG)
    params = jnp.concatenate(
        [gtn, jnp.broadcast_to(jnp.float32(iou_thresh), (_B, 1))], axis=1
    )
    lane = jax.lax.broadcasted_iota(jnp.int32, (1, _B, _NB), 2)
    thr_full = jnp.full((1, _B, _NB), _CONF_THRESH, jnp.float32)
    thr_tail = jnp.where(lane < _TAIL, _CONF_THRESH, jnp.inf)
    thr = jnp.concatenate([thr_full, thr_tail], axis=0)

    det, mp = pl.pallas_call(
        _body,
        grid=(_STEPS,),
        in_specs=[
            pl.BlockSpec((5, _B, _NB), lambda i: (0, 0, i)),
            pl.BlockSpec((1, _B, _NB), lambda i: (0, 0, i)),
            pl.BlockSpec((1, _B, _NB), lambda i: ((i + 1) // _STEPS, 0, 0)),
            pl.BlockSpec((_B, 5), lambda i: (0, 0)),
        ],
        out_specs=[
            pl.BlockSpec((1, 1), lambda i: (0, 0)),
            pl.BlockSpec((_B, 1), lambda i: (0, 0)),
        ],
        out_shape=[
            jax.ShapeDtypeStruct((1, 1), jnp.float32),
            jax.ShapeDtypeStruct((_B, 1), jnp.float32),
        ],
        scratch_shapes=[pltpu.VMEM((_B, _NB), jnp.float32)],
    )(xa, xb, thr, params)
    return det[0, 0], mp[:, 0]


# speedup vs baseline: 1.5037x; 1.5037x over previous
import jax
import jax.numpy as jnp
from jax.experimental import pallas as pl
from jax.experimental.pallas import tpu as pltpu

_FIG = 640.0
_CONF_THRESH = 0.2
_B, _N = 8, 20000
_NB = 2560
_STEPS = (_N + _NB - 1) // _NB
_TAIL = _N - (_STEPS - 1) * _NB
_NEG_INF = float("-inf")


def _body(x_ref, thr_ref, p_ref, det_ref, mp_ref, acc_ref):
    i = pl.program_id(0)

    gx1 = p_ref[:, 0:1]
    gy1 = p_ref[:, 1:2]
    gx2 = p_ref[:, 2:3]
    gy2 = p_ref[:, 3:4]
    tv = p_ref[:, 4:5]
    area2 = (gx2 - gx1) * (gy2 - gy1)

    cx = x_ref[0]
    cy = x_ref[1]
    w = x_ref[2]
    h = x_ref[3]
    conf = x_ref[4]
    clsf = x_ref[6]
    thr = thr_ref[0]

    hw = w * 0.5
    hh = h * 0.5
    w1 = cx - hw
    w2 = cx + hw
    h1 = cy - hh
    h2 = cy + hh
    iw = jnp.maximum(jnp.minimum(w2, gx2) - jnp.maximum(w1, gx1), 0.0)
    ih = jnp.maximum(jnp.minimum(h2, gy2) - jnp.maximum(h1, gy1), 0.0)
    inter = iw * ih
    area1 = (w2 - w1) * (h2 - h1)
    union = area1 + area2 - inter
    m = (inter >= tv * union) & (union > 0.0)
    m = m & (clsf.astype(jnp.int32) == 0) & (conf > thr)
    cand = jnp.where(m, conf, _NEG_INF)

    @pl.when(i == 0)
    def _():
        acc_ref[...] = cand

    @pl.when(i > 0)
    def _():
        acc_ref[...] = jnp.maximum(acc_ref[...], cand)

    @pl.when(i == _STEPS - 1)
    def _():
        mx = jnp.max(acc_ref[...], axis=1, keepdims=True)
        mp = jnp.where(mx == _NEG_INF, 0.0, mx)
        mp_ref[...] = mp
        det_ref[...] = jnp.broadcast_to(jnp.sum(mp) * (1.0 / _B), (1, 1))


def kernel(YOLOoutputs, gt, iou_thresh):
    xt = jnp.transpose(YOLOoutputs, (2, 0, 1))
    gtn = gt * jnp.float32(1.0 / _FIG)
    params = jnp.concatenate(
        [gtn, jnp.broadcast_to(jnp.float32(iou_thresh), (_B, 1))], axis=1
    )
    lane = jax.lax.broadcasted_iota(jnp.int32, (1, _B, _NB), 2)
    thr_full = jnp.full((1, _B, _NB), _CONF_THRESH, jnp.float32)
    thr_tail = jnp.where(lane < _TAIL, _CONF_THRESH, jnp.inf)
    thr = jnp.concatenate([thr_full, thr_tail], axis=0)

    det, mp = pl.pallas_call(
        _body,
        grid=(_STEPS,),
        in_specs=[
            pl.BlockSpec((7, _B, _NB), lambda i: (0, 0, i)),
            pl.BlockSpec((1, _B, _NB), lambda i: ((i + 1) // _STEPS, 0, 0)),
            pl.BlockSpec((_B, 5), lambda i: (0, 0)),
        ],
        out_specs=[
            pl.BlockSpec((1, 1), lambda i: (0, 0)),
            pl.BlockSpec((_B, 1), lambda i: (0, 0)),
        ],
        out_shape=[
            jax.ShapeDtypeStruct((1, 1), jnp.float32),
            jax.ShapeDtypeStruct((_B, 1), jnp.float32),
        ],
        scratch_shapes=[pltpu.VMEM((_B, _NB), jnp.float32)],
    )(xt, thr, params)
    return det[0, 0], mp[:, 0]


# speedup vs baseline: 2.0916x; 1.3910x over previous
import jax
import jax.numpy as jnp
from jax.experimental import pallas as pl
from jax.experimental.pallas import tpu as pltpu

_FIG = 640.0
_CONF_THRESH = 0.2
_B, _N = 8, 20000
_NEG_INF = float("-inf")


def _body(x_ref, p_ref, det_ref, mp_ref):
    gx1 = p_ref[:, 0:1]
    gy1 = p_ref[:, 1:2]
    gx2 = p_ref[:, 2:3]
    gy2 = p_ref[:, 3:4]
    tv = p_ref[:, 4:5]
    area2 = (gx2 - gx1) * (gy2 - gy1)

    cx = x_ref[0]
    cy = x_ref[1]
    w = x_ref[2]
    h = x_ref[3]
    conf = x_ref[4]
    clsf = x_ref[6]

    hw = w * 0.5
    hh = h * 0.5
    w1 = cx - hw
    w2 = cx + hw
    h1 = cy - hh
    h2 = cy + hh
    iw = jnp.maximum(jnp.minimum(w2, gx2) - jnp.maximum(w1, gx1), 0.0)
    ih = jnp.maximum(jnp.minimum(h2, gy2) - jnp.maximum(h1, gy1), 0.0)
    inter = iw * ih
    area1 = (w2 - w1) * (h2 - h1)
    union = area1 + area2 - inter
    m = (inter >= tv * union) & (union > 0.0)
    m = m & (clsf.astype(jnp.int32) == 0) & (conf > _CONF_THRESH)
    cand = jnp.where(m, conf, _NEG_INF)

    mx = jnp.max(cand, axis=1, keepdims=True)
    mp = jnp.where(mx == _NEG_INF, 0.0, mx)
    mp_ref[...] = mp
    det_ref[...] = jnp.broadcast_to(jnp.sum(mp) * (1.0 / _B), (1, 1))


def kernel(YOLOoutputs, gt, iou_thresh):
    xt = jnp.transpose(YOLOoutputs, (2, 0, 1))
    gtn = gt * jnp.float32(1.0 / _FIG)
    params = jnp.concatenate(
        [gtn, jnp.broadcast_to(jnp.float32(iou_thresh), (_B, 1))], axis=1
    )
    det, mp = pl.pallas_call(
        _body,
        in_specs=[
            pl.BlockSpec(memory_space=pltpu.MemorySpace.VMEM),
            pl.BlockSpec(memory_space=pltpu.MemorySpace.VMEM),
        ],
        out_specs=[
            pl.BlockSpec(memory_space=pltpu.MemorySpace.VMEM),
            pl.BlockSpec(memory_space=pltpu.MemorySpace.VMEM),
        ],
        out_shape=[
            jax.ShapeDtypeStruct((1, 1), jnp.float32),
            jax.ShapeDtypeStruct((_B, 1), jnp.float32),
        ],
    )(xt, params)
    return det[0, 0], mp[:, 0]
